# baseline (speedup 1.0000x reference)
import jax
import jax.numpy as jnp
from jax.experimental import pallas as pl
from jax.experimental.pallas import tpu as pltpu

_NB = 128
_NCLS = 10


def _cnn_kernel(x_ref, a1_ref, a2_ref, a3_ref, wf1_ref, wf2_ref,
                b1_ref, b2_ref, b3_ref, bf1_ref, bf2_ref, out_ref):
    f32 = jnp.float32
    nb = x_ref.shape[1]

    def shift_cat(p, width):
        z = jnp.zeros((1, nb, width), f32)
        up = jnp.concatenate([z, p[:-1]], axis=0)
        dn = jnp.concatenate([p[1:], z], axis=0)
        return jnp.concatenate([up, p, dn], axis=-1)

    def pool(o, hh):
        m = jnp.maximum(o[:, :256], o[:, 256:])
        m = m.reshape(hh // 2, 2, nb, 256)
        return jnp.maximum(m[:, 0], m[:, 1])

    x1 = shift_cat(x_ref[...], 96).reshape(32 * nb, 288)
    o1 = jnp.dot(x1, a1_ref[...], preferred_element_type=f32)
    o1 = jnp.maximum(o1 + b1_ref[...], 0.0)
    p1 = pool(o1, 32)

    x2 = shift_cat(p1, 256).reshape(16 * nb, 768)
    o2 = jnp.dot(x2, a2_ref[...], preferred_element_type=f32)
    o2 = jnp.maximum(o2 + b2_ref[...], 0.0)
    p2 = pool(o2, 16)

    x3 = shift_cat(p2, 256).reshape(8 * nb, 768)
    o3 = jnp.dot(x3, a3_ref[...], preferred_element_type=f32)
    o3 = jnp.maximum(o3 + b3_ref[...], 0.0)
    p3 = pool(o3, 8)

    h = bf1_ref[...]
    for i in range(4):
        h = h + jnp.dot(p3[i], wf1_ref[i * 256:(i + 1) * 256, :],
                        preferred_element_type=f32)
    h = jnp.maximum(h, 0.0)

    out = jnp.dot(h, wf2_ref[...], preferred_element_type=f32) + bf2_ref[...]
    out_ref[...] = out.astype(out_ref.dtype)


def _banded(wc, w_n, ci_major=False):
    v = jnp.arange(3)[:, None, None]
    wi = jnp.arange(w_n)[None, :, None]
    w = jnp.arange(w_n)[None, None, :]
    t = (wi == w + v - 1).astype(wc.dtype)
    tm = jnp.transpose(t, (1, 2, 0)).reshape(w_n * w_n, 3)
    cin, cout = wc.shape[2], wc.shape[3]
    blocks = []
    for u in range(3):
        a = jnp.dot(tm, wc[u].reshape(3, cin * cout))
        a = a.reshape(w_n, w_n, cin, cout)
        a = jnp.transpose(a, (2, 0, 1, 3) if ci_major else (0, 2, 1, 3))
        a = a.reshape(w_n * cin, w_n // 2, 2, cout)
        a = jnp.transpose(a, (0, 2, 1, 3)).reshape(w_n * cin, w_n * cout)
        blocks.append(a)
    return jnp.concatenate(blocks, axis=0)


def _forward(x_nchw, w_c1, b_c1, w_c2, b_c2, w_c3, b_c3,
             w_fc1, b_fc1, w_fc2, b_fc2):
    n = x_nchw.shape[0]
    f32 = jnp.float32

    x = jnp.transpose(x_nchw, (2, 0, 1, 3)).reshape(32, n, 96)

    a1 = _banded(w_c1, 32, ci_major=True)
    a2 = _banded(w_c2, 16)
    a3 = _banded(w_c3, 8)

    wf1p = jnp.transpose(w_fc1.reshape(64, 16, 500), (1, 0, 2))
    wf1c = jnp.pad(wf1p.reshape(1024, 500), ((0, 0), (0, 12)))

    wf2p = jnp.pad(w_fc2, ((0, 12), (0, 0)))

    b1t = jnp.tile(b_c1.reshape(-1), 32).reshape(1, 512)
    b2t = jnp.tile(b_c2.reshape(-1), 16).reshape(1, 512)
    b3t = jnp.tile(b_c3.reshape(-1), 8).reshape(1, 512)
    bf1p = jnp.pad(b_fc1.reshape(-1), (0, 12)).reshape(1, 512)
    bf2r = b_fc2.reshape(1, _NCLS)

    nb = _NB
    n_pad = (-n) % nb
    if n_pad:
        x = jnp.pad(x, ((0, 0), (0, n_pad), (0, 0)))
    ng = (n + n_pad) // nb

    const = lambda i: (0, 0)
    out = pl.pallas_call(
        _cnn_kernel,
        out_shape=jax.ShapeDtypeStruct((n + n_pad, _NCLS), x_nchw.dtype),
        grid=(ng,),
        in_specs=[
            pl.BlockSpec((32, nb, 96), lambda i: (0, i, 0)),
            pl.BlockSpec((288, 512), const),
            pl.BlockSpec((768, 512), const),
            pl.BlockSpec((768, 512), const),
            pl.BlockSpec((1024, 512), const),
            pl.BlockSpec((512, _NCLS), const),
            pl.BlockSpec((1, 512), const),
            pl.BlockSpec((1, 512), const),
            pl.BlockSpec((1, 512), const),
            pl.BlockSpec((1, 512), const),
            pl.BlockSpec((1, _NCLS), const),
        ],
        out_specs=pl.BlockSpec((nb, _NCLS), lambda i: (i, 0)),
        compiler_params=pltpu.CompilerParams(
            dimension_semantics=("parallel",),
            vmem_limit_bytes=100 * 1024 * 1024,
        ),
    )(x, a1, a2, a3, wf1c, wf2p, b1t, b2t, b3t, bf1p, bf2r)
    return out[:n] if n_pad else out


def kernel(x_nchw, w_c1, b_c1, w_c2, b_c2, w_c3, b_c3,
           w_fc1, b_fc1, w_fc2, b_fc2):
    args = (x_nchw, w_c1, b_c1, w_c2, b_c2, w_c3, b_c3,
            w_fc1, b_fc1, w_fc2, b_fc2)
    devs = jax.devices()
    n = x_nchw.shape[0]
    if len(devs) >= 2 and n % (2 * _NB) == 0:
        mesh = jax.sharding.Mesh(devs[:2], ("d",))
        p = jax.sharding.PartitionSpec
        f = jax.shard_map(_forward, mesh=mesh,
                          in_specs=(p("d"),) + (p(),) * 10,
                          out_specs=p("d"), check_vma=False)
        return f(*args)
    return _forward(*args)


# speedup vs baseline: 4.3924x; 4.3924x over previous
import jax
import jax.numpy as jnp
from jax.experimental import pallas as pl
from jax.experimental.pallas import tpu as pltpu

_NB = 128
_NCLS = 10


def _cnn_kernel(x_ref, a1_ref, a2_ref, a3_ref, wf1_ref, wf2_ref,
                b1_ref, b2_ref, b3_ref, bf1_ref, bf2_ref, out_ref):
    bf16 = jnp.bfloat16
    f32 = jnp.float32
    nb = x_ref.shape[1]

    def shift_cat(p, width):
        z = jnp.zeros((1, nb, width), p.dtype)
        up = jnp.concatenate([z, p[:-1]], axis=0)
        dn = jnp.concatenate([p[1:], z], axis=0)
        return jnp.concatenate([up, p, dn], axis=-1)

    def relu_pool(o, b, hh):
        m = jnp.maximum(o + b, 0.0).astype(bf16)
        m = jnp.maximum(m[:, :256], m[:, 256:])
        m = m.reshape(hh // 2, 2, nb, 256)
        return jnp.maximum(m[:, 0], m[:, 1])

    x1 = shift_cat(x_ref[...], 96).reshape(32 * nb, 288)
    o1 = jnp.dot(x1, a1_ref[...], preferred_element_type=f32)
    p1 = relu_pool(o1, b1_ref[...], 32)

    x2 = shift_cat(p1, 256).reshape(16 * nb, 768)
    o2 = jnp.dot(x2, a2_ref[...], preferred_element_type=f32)
    p2 = relu_pool(o2, b2_ref[...], 16)

    x3 = shift_cat(p2, 256).reshape(8 * nb, 768)
    o3 = jnp.dot(x3, a3_ref[...], preferred_element_type=f32)
    p3 = relu_pool(o3, b3_ref[...], 8)

    h = bf1_ref[...]
    for i in range(4):
        h = h + jnp.dot(p3[i], wf1_ref[i * 256:(i + 1) * 256, :],
                        preferred_element_type=f32)
    h = jnp.maximum(h, 0.0).astype(bf16)

    out = jnp.dot(h, wf2_ref[...], preferred_element_type=f32) + bf2_ref[...]
    out_ref[...] = out.astype(out_ref.dtype)


def _banded(wc, w_n, ci_major=False):
    v = jnp.arange(3)[:, None, None]
    wi = jnp.arange(w_n)[None, :, None]
    w = jnp.arange(w_n)[None, None, :]
    t = (wi == w + v - 1).astype(wc.dtype)
    tm = jnp.transpose(t, (1, 2, 0)).reshape(w_n * w_n, 3)
    cin, cout = wc.shape[2], wc.shape[3]
    blocks = []
    for u in range(3):
        a = jnp.dot(tm, wc[u].reshape(3, cin * cout))
        a = a.reshape(w_n, w_n, cin, cout)
        a = jnp.transpose(a, (2, 0, 1, 3) if ci_major else (0, 2, 1, 3))
        a = a.reshape(w_n * cin, w_n // 2, 2, cout)
        a = jnp.transpose(a, (0, 2, 1, 3)).reshape(w_n * cin, w_n * cout)
        blocks.append(a)
    return jnp.concatenate(blocks, axis=0)


@jax.jit
def _forward(x_nchw, w_c1, b_c1, w_c2, b_c2, w_c3, b_c3,
             w_fc1, b_fc1, w_fc2, b_fc2):
    n = x_nchw.shape[0]
    f32 = jnp.float32

    bf16 = jnp.bfloat16
    x = jnp.transpose(x_nchw.astype(bf16), (2, 0, 1, 3)).reshape(32, n, 96)

    a1 = _banded(w_c1, 32, ci_major=True).astype(bf16)
    a2 = _banded(w_c2, 16).astype(bf16)
    a3 = _banded(w_c3, 8).astype(bf16)

    wf1p = jnp.transpose(w_fc1.reshape(64, 16, 500), (1, 0, 2))
    wf1c = jnp.pad(wf1p.reshape(1024, 500), ((0, 0), (0, 12))).astype(bf16)

    wf2p = jnp.pad(w_fc2, ((0, 12), (0, 0))).astype(bf16)

    b1t = jnp.tile(b_c1.reshape(-1), 32).reshape(1, 512)
    b2t = jnp.tile(b_c2.reshape(-1), 16).reshape(1, 512)
    b3t = jnp.tile(b_c3.reshape(-1), 8).reshape(1, 512)
    bf1p = jnp.pad(b_fc1.reshape(-1), (0, 12)).reshape(1, 512)
    bf2r = b_fc2.reshape(1, _NCLS)

    nb = _NB
    n_pad = (-n) % nb
    if n_pad:
        x = jnp.pad(x, ((0, 0), (0, n_pad), (0, 0)))
    ng = (n + n_pad) // nb

    const = lambda i: (0, 0)
    out = pl.pallas_call(
        _cnn_kernel,
        out_shape=jax.ShapeDtypeStruct((n + n_pad, _NCLS), x_nchw.dtype),
        grid=(ng,),
        in_specs=[
            pl.BlockSpec((32, nb, 96), lambda i: (0, i, 0)),
            pl.BlockSpec((288, 512), const),
            pl.BlockSpec((768, 512), const),
            pl.BlockSpec((768, 512), const),
            pl.BlockSpec((1024, 512), const),
            pl.BlockSpec((512, _NCLS), const),
            pl.BlockSpec((1, 512), const),
            pl.BlockSpec((1, 512), const),
            pl.BlockSpec((1, 512), const),
            pl.BlockSpec((1, 512), const),
            pl.BlockSpec((1, _NCLS), const),
        ],
        out_specs=pl.BlockSpec((nb, _NCLS), lambda i: (i, 0)),
        compiler_params=pltpu.CompilerParams(
            dimension_semantics=("parallel",),
            vmem_limit_bytes=100 * 1024 * 1024,
        ),
    )(x, a1, a2, a3, wf1c, wf2p, b1t, b2t, b3t, bf1p, bf2r)
    return out[:n] if n_pad else out


def kernel(x_nchw, w_c1, b_c1, w_c2, b_c2, w_c3, b_c3,
           w_fc1, b_fc1, w_fc2, b_fc2):
    return _forward(x_nchw, w_c1, b_c1, w_c2, b_c2, w_c3, b_c3,
                    w_fc1, b_fc1, w_fc2, b_fc2)


# speedup vs baseline: 4.4665x; 1.0169x over previous
import jax
import jax.numpy as jnp
from jax.experimental import pallas as pl
from jax.experimental.pallas import tpu as pltpu

_NB = 256
_NCLS = 10


def _cnn_kernel(x_ref, a1_ref, a2_ref, a3_ref, wf1_ref, wf2_ref,
                b1_ref, b2_ref, b3_ref, bf1_ref, bf2_ref, out_ref):
    bf16 = jnp.bfloat16
    f32 = jnp.float32
    nb = x_ref.shape[1]

    def shift_cat(p, width):
        z = jnp.zeros((1, nb, width), p.dtype)
        up = jnp.concatenate([z, p[:-1]], axis=0)
        dn = jnp.concatenate([p[1:], z], axis=0)
        return jnp.concatenate([up, p, dn], axis=-1)

    def relu_pool(o, b, hh):
        m = jnp.maximum(o + b, 0.0).astype(bf16)
        m = jnp.maximum(m[:, :256], m[:, 256:])
        m = m.reshape(hh // 2, 2, nb, 256)
        return jnp.maximum(m[:, 0], m[:, 1])

    x1 = shift_cat(x_ref[...], 96).reshape(32 * nb, 288)
    o1 = jnp.dot(x1, a1_ref[...], preferred_element_type=f32)
    p1 = relu_pool(o1, b1_ref[...], 32)

    x2 = shift_cat(p1, 256).reshape(16 * nb, 768)
    o2 = jnp.dot(x2, a2_ref[...], preferred_element_type=f32)
    p2 = relu_pool(o2, b2_ref[...], 16)

    x3 = shift_cat(p2, 256).reshape(8 * nb, 768)
    o3 = jnp.dot(x3, a3_ref[...], preferred_element_type=f32)
    p3 = relu_pool(o3, b3_ref[...], 8)

    h = bf1_ref[...]
    for i in range(4):
        h = h + jnp.dot(p3[i], wf1_ref[i * 256:(i + 1) * 256, :],
                        preferred_element_type=f32)
    h = jnp.maximum(h, 0.0).astype(bf16)

    out = jnp.dot(h, wf2_ref[...], preferred_element_type=f32) + bf2_ref[...]
    out_ref[...] = out.astype(out_ref.dtype)


def _banded(wc, w_n, ci_major=False):
    v = jnp.arange(3)[:, None, None]
    wi = jnp.arange(w_n)[None, :, None]
    w = jnp.arange(w_n)[None, None, :]
    t = (wi == w + v - 1).astype(wc.dtype)
    tm = jnp.transpose(t, (1, 2, 0)).reshape(w_n * w_n, 3)
    cin, cout = wc.shape[2], wc.shape[3]
    blocks = []
    for u in range(3):
        a = jnp.dot(tm, wc[u].reshape(3, cin * cout))
        a = a.reshape(w_n, w_n, cin, cout)
        a = jnp.transpose(a, (2, 0, 1, 3) if ci_major else (0, 2, 1, 3))
        a = a.reshape(w_n * cin, w_n // 2, 2, cout)
        a = jnp.transpose(a, (0, 2, 1, 3)).reshape(w_n * cin, w_n * cout)
        blocks.append(a)
    return jnp.concatenate(blocks, axis=0)


@jax.jit
def _forward(x_nchw, w_c1, b_c1, w_c2, b_c2, w_c3, b_c3,
             w_fc1, b_fc1, w_fc2, b_fc2):
    n = x_nchw.shape[0]
    f32 = jnp.float32

    bf16 = jnp.bfloat16
    x = jnp.transpose(x_nchw.astype(bf16), (2, 0, 1, 3)).reshape(32, n, 96)

    a1 = _banded(w_c1, 32, ci_major=True).astype(bf16)
    a2 = _banded(w_c2, 16).astype(bf16)
    a3 = _banded(w_c3, 8).astype(bf16)

    wf1p = jnp.transpose(w_fc1.reshape(64, 16, 500), (1, 0, 2))
    wf1c = jnp.pad(wf1p.reshape(1024, 500), ((0, 0), (0, 12))).astype(bf16)

    wf2p = jnp.pad(w_fc2, ((0, 12), (0, 0))).astype(bf16)

    b1t = jnp.tile(b_c1.reshape(-1), 32).reshape(1, 512)
    b2t = jnp.tile(b_c2.reshape(-1), 16).reshape(1, 512)
    b3t = jnp.tile(b_c3.reshape(-1), 8).reshape(1, 512)
    bf1p = jnp.pad(b_fc1.reshape(-1), (0, 12)).reshape(1, 512)
    bf2r = b_fc2.reshape(1, _NCLS)

    nb = _NB
    n_pad = (-n) % nb
    if n_pad:
        x = jnp.pad(x, ((0, 0), (0, n_pad), (0, 0)))
    ng = (n + n_pad) // nb

    const = lambda i: (0, 0)
    out = pl.pallas_call(
        _cnn_kernel,
        out_shape=jax.ShapeDtypeStruct((n + n_pad, _NCLS), x_nchw.dtype),
        grid=(ng,),
        in_specs=[
            pl.BlockSpec((32, nb, 96), lambda i: (0, i, 0)),
            pl.BlockSpec((288, 512), const),
            pl.BlockSpec((768, 512), const),
            pl.BlockSpec((768, 512), const),
            pl.BlockSpec((1024, 512), const),
            pl.BlockSpec((512, _NCLS), const),
            pl.BlockSpec((1, 512), const),
            pl.BlockSpec((1, 512), const),
            pl.BlockSpec((1, 512), const),
            pl.BlockSpec((1, 512), const),
            pl.BlockSpec((1, _NCLS), const),
        ],
        out_specs=pl.BlockSpec((nb, _NCLS), lambda i: (i, 0)),
        compiler_params=pltpu.CompilerParams(
            dimension_semantics=("parallel",),
            vmem_limit_bytes=100 * 1024 * 1024,
        ),
    )(x, a1, a2, a3, wf1c, wf2p, b1t, b2t, b3t, bf1p, bf2r)
    return out[:n] if n_pad else out


def kernel(x_nchw, w_c1, b_c1, w_c2, b_c2, w_c3, b_c3,
           w_fc1, b_fc1, w_fc2, b_fc2):
    return _forward(x_nchw, w_c1, b_c1, w_c2, b_c2, w_c3, b_c3,
                    w_fc1, b_fc1, w_fc2, b_fc2)
